# baseline (speedup 1.0000x reference)
import functools

import jax
import jax.numpy as jnp
from jax import lax
from jax.experimental import pallas as pl
from jax.experimental.pallas import tpu as pltpu
from jax.experimental.pallas import tpu_sc as plsc


@functools.partial(jax.jit, static_argnums=())
def _gather_rows(table, idx):
    V, D = table.shape
    (N,) = idx.shape
    info = plsc.get_sparse_core_info()
    NC, NS = info.num_cores, info.num_subcores
    NW = NC * NS
    b_per_w = N // NW
    CHUNK = 32
    NBUF = 5
    GA = 2
    NCHUNK = b_per_w // CHUNK

    mesh = plsc.VectorSubcoreMesh(core_axis_name="c", subcore_axis_name="s")

    @functools.partial(
        pl.kernel,
        mesh=mesh,
        out_type=jax.ShapeDtypeStruct((N, D), jnp.float32),
        scratch_types=[
            pltpu.VMEM((b_per_w,), jnp.int32),
        ]
        + [pltpu.VMEM((CHUNK, D), jnp.float32)] * NBUF
        + [pltpu.SemaphoreType.DMA] * (2 * NBUF),
    )
    def k(table_hbm, idx_hbm, out_hbm, idx_v, *bufs_sems):
        bufs = bufs_sems[:NBUF]
        gsems = bufs_sems[NBUF : 2 * NBUF]
        osems = bufs_sems[2 * NBUF : 3 * NBUF]
        wid = lax.axis_index("s") * NC + lax.axis_index("c")
        base = wid * b_per_w
        pltpu.sync_copy(idx_hbm.at[pl.ds(base, b_per_w)], idx_v)

        def gather(c):
            return pltpu.async_copy(
                table_hbm.at[idx_v.at[pl.ds(c * CHUNK, CHUNK)]],
                bufs[c % NBUF],
                gsems[c % NBUF],
            )

        def writeout(c):
            return pltpu.async_copy(
                bufs[c % NBUF],
                out_hbm.at[pl.ds(base + c * CHUNK, CHUNK)],
                osems[c % NBUF],
            )

        gcp = {c: gather(c) for c in range(min(GA, NCHUNK))}
        wcp = {}
        for c in range(NCHUNK):
            gcp[c].wait()
            wcp[c] = writeout(c)
            nxt = c + GA
            if nxt < NCHUNK:
                prev = nxt - NBUF
                if prev >= 0:
                    wcp[prev].wait()
                gcp[nxt] = gather(nxt)
        for c in range(max(0, NCHUNK - NBUF), NCHUNK):
            wcp[c].wait()

    return k(table, idx)


def kernel(input_ids, label, attention_mask, table):
    B, S = input_ids.shape
    V, D = table.shape
    idx = input_ids.reshape(B * S).astype(jnp.int32)
    out = _gather_rows(table, idx)
    return (out.reshape(B, S, D), label, attention_mask)


# speedup vs baseline: 1.0244x; 1.0244x over previous
import functools

import jax
import jax.numpy as jnp
from jax import lax
from jax.experimental import pallas as pl
from jax.experimental.pallas import tpu as pltpu
from jax.experimental.pallas import tpu_sc as plsc


@functools.partial(jax.jit, static_argnums=())
def _gather_rows(table, idx):
    V, D = table.shape
    (N,) = idx.shape
    info = plsc.get_sparse_core_info()
    NC, NS = info.num_cores, info.num_subcores
    NW = NC * NS
    b_per_w = N // NW
    CHUNK = 16
    NBUF = 10
    GA = 6
    NCHUNK = b_per_w // CHUNK

    mesh = plsc.VectorSubcoreMesh(core_axis_name="c", subcore_axis_name="s")

    @functools.partial(
        pl.kernel,
        mesh=mesh,
        out_type=jax.ShapeDtypeStruct((N, D), jnp.float32),
        scratch_types=[
            pltpu.VMEM((b_per_w,), jnp.int32),
        ]
        + [pltpu.VMEM((CHUNK, D), jnp.float32)] * NBUF
        + [pltpu.SemaphoreType.DMA] * (2 * NBUF),
    )
    def k(table_hbm, idx_hbm, out_hbm, idx_v, *bufs_sems):
        bufs = bufs_sems[:NBUF]
        gsems = bufs_sems[NBUF : 2 * NBUF]
        osems = bufs_sems[2 * NBUF : 3 * NBUF]
        wid = lax.axis_index("s") * NC + lax.axis_index("c")
        base = wid * b_per_w
        pltpu.sync_copy(idx_hbm.at[pl.ds(base, b_per_w)], idx_v)

        def gather(c):
            return pltpu.async_copy(
                table_hbm.at[idx_v.at[pl.ds(c * CHUNK, CHUNK)]],
                bufs[c % NBUF],
                gsems[c % NBUF],
            )

        def writeout(c):
            return pltpu.async_copy(
                bufs[c % NBUF],
                out_hbm.at[pl.ds(base + c * CHUNK, CHUNK)],
                osems[c % NBUF],
            )

        gcp = {c: gather(c) for c in range(min(GA, NCHUNK))}
        wcp = {}
        for c in range(NCHUNK):
            gcp[c].wait()
            wcp[c] = writeout(c)
            nxt = c + GA
            if nxt < NCHUNK:
                prev = nxt - NBUF
                if prev >= 0:
                    wcp[prev].wait()
                gcp[nxt] = gather(nxt)
        for c in range(max(0, NCHUNK - NBUF), NCHUNK):
            wcp[c].wait()

    return k(table, idx)


def kernel(input_ids, label, attention_mask, table):
    B, S = input_ids.shape
    V, D = table.shape
    idx = input_ids.reshape(B * S).astype(jnp.int32)
    out = _gather_rows(table, idx)
    return (out.reshape(B, S, D), label, attention_mask)


# speedup vs baseline: 1.0374x; 1.0128x over previous
import functools

import jax
import jax.numpy as jnp
from jax import lax
from jax.experimental import pallas as pl
from jax.experimental.pallas import tpu as pltpu
from jax.experimental.pallas import tpu_sc as plsc


@functools.partial(jax.jit, static_argnums=())
def _gather_rows(table, idx):
    V, D = table.shape
    B, S = idx.shape
    N = B * S
    info = plsc.get_sparse_core_info()
    NC, NS = info.num_cores, info.num_subcores
    NW = NC * NS
    b_per_w = N // NW
    blocks_per_row = S // b_per_w
    CHUNK = 32
    NBUF = 5
    GA = 4
    NCHUNK = b_per_w // CHUNK

    mesh = plsc.VectorSubcoreMesh(core_axis_name="c", subcore_axis_name="s")

    @functools.partial(
        pl.kernel,
        mesh=mesh,
        out_type=jax.ShapeDtypeStruct((N, D), jnp.float32),
        scratch_types=[
            pltpu.VMEM((b_per_w,), jnp.int32),
        ]
        + [pltpu.VMEM((CHUNK, D), jnp.float32)] * NBUF
        + [pltpu.SemaphoreType.DMA] * (2 * NBUF),
    )
    def k(table_hbm, idx_hbm, out_hbm, idx_v, *bufs_sems):
        bufs = bufs_sems[:NBUF]
        gsems = bufs_sems[NBUF : 2 * NBUF]
        osems = bufs_sems[2 * NBUF : 3 * NBUF]
        wid = lax.axis_index("s") * NC + lax.axis_index("c")
        base = wid * b_per_w
        pltpu.sync_copy(
            idx_hbm.at[wid // blocks_per_row, pl.ds((wid % blocks_per_row) * b_per_w, b_per_w)],
            idx_v,
        )

        def gather(c):
            return pltpu.async_copy(
                table_hbm.at[idx_v.at[pl.ds(c * CHUNK, CHUNK)]],
                bufs[c % NBUF],
                gsems[c % NBUF],
            )

        def writeout(c):
            return pltpu.async_copy(
                bufs[c % NBUF],
                out_hbm.at[pl.ds(base + c * CHUNK, CHUNK)],
                osems[c % NBUF],
            )

        gcp = {c: gather(c) for c in range(min(GA, NCHUNK))}
        wcp = {}
        for c in range(NCHUNK):
            gcp[c].wait()
            wcp[c] = writeout(c)
            nxt = c + GA
            if nxt < NCHUNK:
                prev = nxt - NBUF
                if prev >= 0:
                    wcp[prev].wait()
                gcp[nxt] = gather(nxt)
        for c in range(max(0, NCHUNK - NBUF), NCHUNK):
            wcp[c].wait()

    return k(table, idx)


def kernel(input_ids, label, attention_mask, table):
    B, S = input_ids.shape
    V, D = table.shape
    out = _gather_rows(table, input_ids.astype(jnp.int32))
    return (out.reshape(B, S, D), label, attention_mask)
